# baseline (speedup 1.0000x reference)
import functools

import jax
import jax.numpy as jnp
from jax import lax
from jax.experimental import pallas as pl
from jax.experimental.pallas import tpu as pltpu
from jax.experimental.pallas import tpu_sc as plsc

B = 16384
D = 64
D2 = 2 * D
NC = 2
NS = 16
NW = NC * NS
BPW = B // NW
CHUNK = 64
NCHUNK = BPW // CHUNK


def _sc_body(rows_u, rows_i, t_eu, t_mu, t_ei, t_mi, t_fu, t_fi,
             o_eu, o_mu, o_ei, o_mi, o_fu, o_fi,
             idx_u, idx_i, bufs, sg, sw):
    wid = lax.axis_index("s") * NC + lax.axis_index("c")
    tabs = (t_eu, t_mu, t_ei, t_mi, t_fu, t_fi)
    outs = (o_eu, o_mu, o_ei, o_mi, o_fu, o_fi)

    def fire_gathers(ch, s):
        base = wid * BPW + ch * CHUNK
        pltpu.sync_copy(rows_u.at[pl.ds(base, CHUNK)], idx_u.at[s])
        pltpu.sync_copy(rows_i.at[pl.ds(base, CHUNK)], idx_i.at[s])
        use_u = (1, 1, 0, 0, 1, 0)
        cps = []
        for t, tab in enumerate(tabs):
            idx = idx_u.at[s] if use_u[t] else idx_i.at[s]
            cps.append(pltpu.async_copy(tab.at[idx], bufs[s][t], sg[s]))
        return cps

    def fire_writes(ch, s):
        base = wid * BPW + ch * CHUNK
        return [pltpu.async_copy(bufs[s][t], outs[t].at[pl.ds(base, CHUNK)],
                                 sw[s])
                for t in range(6)]

    pend_g = {0: fire_gathers(0, 0)}
    pend_w = {}
    for ch in range(NCHUNK):
        s = ch % 2
        if ch + 1 < NCHUNK:
            o = (ch + 1) % 2
            for cp in pend_w.pop(o, ()):
                cp.wait()
            pend_g[o] = fire_gathers(ch + 1, o)
        for cp in pend_g.pop(s):
            cp.wait()
        pend_w[s] = fire_writes(ch, s)
    for cps in pend_w.values():
        for cp in cps:
            cp.wait()


@functools.cache
def _sc_gather():
    return pl.kernel(
        _sc_body,
        out_type=[jax.ShapeDtypeStruct((B, D2), jnp.float32),
                  jax.ShapeDtypeStruct((B, D2), jnp.int32),
                  jax.ShapeDtypeStruct((B, D2), jnp.float32),
                  jax.ShapeDtypeStruct((B, D2), jnp.int32),
                  jax.ShapeDtypeStruct((B, D2), jnp.float32),
                  jax.ShapeDtypeStruct((B, D2), jnp.float32)],
        mesh=plsc.VectorSubcoreMesh(core_axis_name="c", subcore_axis_name="s"),
        scratch_types=[
            pltpu.VMEM((2, CHUNK), jnp.int32),
            pltpu.VMEM((2, CHUNK), jnp.int32),
            tuple(tuple(pltpu.VMEM((CHUNK, D2),
                                   jnp.int32 if t in (1, 3) else jnp.float32)
                        for t in range(6)) for _ in range(2)),
            (pltpu.SemaphoreType.DMA, pltpu.SemaphoreType.DMA),
            (pltpu.SemaphoreType.DMA, pltpu.SemaphoreType.DMA),
        ],
    )


BT = 2048


def _tc_body(u, i, eu2, mu2, ei2, mi2, fu2, fi2,
             w1a, w1b, b1, w2a, w2b, b2, out):
    pu = (u[...] % 2) == 1
    pi = (i[...] % 2) == 1

    def sel(x2, p):
        return jnp.where(p, x2[:, D:], x2[:, :D])

    xu = sel(eu2[...], pu) * sel(mu2[...], pu).astype(jnp.float32)
    xi = sel(ei2[...], pi) * sel(mi2[...], pi).astype(jnp.float32)
    mf = sel(fu2[...], pu) * sel(fi2[...], pi)
    h = jnp.dot(xu, w1a[...], preferred_element_type=jnp.float32)
    h = h + jnp.dot(xi, w1b[...], preferred_element_type=jnp.float32)
    h = jnp.maximum(h + b1[...], 0.0)
    o = jnp.dot(h, w2a[...], preferred_element_type=jnp.float32)
    o = o + jnp.dot(mf, w2b[...], preferred_element_type=jnp.float32)
    out[...] = o + b2[0, 0]


_tc_call = pl.pallas_call(
    _tc_body,
    grid=(B // BT,),
    in_specs=[
        pl.BlockSpec((BT, 1), lambda n: (n, 0)),
        pl.BlockSpec((BT, 1), lambda n: (n, 0)),
        pl.BlockSpec((BT, D2), lambda n: (n, 0)),
        pl.BlockSpec((BT, D2), lambda n: (n, 0)),
        pl.BlockSpec((BT, D2), lambda n: (n, 0)),
        pl.BlockSpec((BT, D2), lambda n: (n, 0)),
        pl.BlockSpec((BT, D2), lambda n: (n, 0)),
        pl.BlockSpec((BT, D2), lambda n: (n, 0)),
        pl.BlockSpec((D, D), lambda n: (0, 0)),
        pl.BlockSpec((D, D), lambda n: (0, 0)),
        pl.BlockSpec((1, D), lambda n: (0, 0)),
        pl.BlockSpec((D, 1), lambda n: (0, 0)),
        pl.BlockSpec((D, 1), lambda n: (0, 0)),
        pl.BlockSpec((1, 1), lambda n: (0, 0)),
    ],
    out_specs=pl.BlockSpec((BT, 1), lambda n: (n, 0)),
    out_shape=jax.ShapeDtypeStruct((B, 1), jnp.float32),
)


def kernel(users, items, emb_user_mlp, emb_item_mlp, emb_user_mf, emb_item_mf,
           user_mask, item_mask, W1, b1, W2, b2):
    half = lambda t: jnp.reshape(t, (t.shape[0] // 2, D2))
    eu2, mu2, ei2, mi2, fu2, fi2 = _sc_gather()(
        users >> 1, items >> 1,
        half(emb_user_mlp), half(user_mask), half(emb_item_mlp),
        half(item_mask), half(emb_user_mf), half(emb_item_mf))
    logits = _tc_call(users.reshape(B, 1), items.reshape(B, 1),
                      eu2, mu2, ei2, mi2, fu2, fi2,
                      W1[:D], W1[D:], b1.reshape(1, D),
                      W2[:D], W2[D:], b2.reshape(1, 1))
    return logits


# speedup vs baseline: 1.8138x; 1.8138x over previous
import functools

import jax
import jax.numpy as jnp
from jax import lax
from jax.experimental import pallas as pl
from jax.experimental.pallas import tpu as pltpu
from jax.experimental.pallas import tpu_sc as plsc

B = 16384
D = 64
N_ROWS = 100000
NC = 2
NS = 16
NW = NC * NS
JOBS = 12
OC = 1024
NCH = B // OC
OSUB = 8


def _sc_body(users, items, t_eu, t_mu, t_fu, t_ei, t_mi, t_fi,
             o_eu, o_mu, o_fu, o_ei, o_mi, o_fi,
             idx_vm, row_vm, out_vm, sem_out):
    wid = lax.axis_index("s") * NC + lax.axis_index("c")

    def side_body(w, idx_src, tabs, outs):
        pltpu.sync_copy(idx_src, idx_vm)
        for j in range(JOBS):
            tab = tabs[j // 4]
            out = outs[j // 4]
            d = w * 4 + (j % 4)
            pltpu.sync_copy(tab.at[d], row_vm)

            def chunk(c, carry):
                sub = lax.rem(c, OSUB)

                @pl.when(c >= OSUB)
                def _():
                    pltpu.make_async_copy(
                        out_vm.at[0], out.at[d, pl.ds(0, OC)], sem_out).wait()

                def gat(k, carry2):
                    iv = idx_vm[pl.ds((c * (OC // 16) + k) * 16, 16)]
                    out_vm[sub, pl.ds(k * 16, 16)] = plsc.load_gather(
                        row_vm, [iv])
                    return carry2

                lax.fori_loop(0, OC // 16, gat, 0)
                pltpu.async_copy(out_vm.at[sub],
                                 out.at[d, pl.ds(c * OC, OC)], sem_out)
                return carry

            lax.fori_loop(0, NCH, chunk, 0)
            for _ in range(OSUB):
                pltpu.make_async_copy(
                    out_vm.at[0], out.at[d, pl.ds(0, OC)], sem_out).wait()

    @pl.when(wid < 16)
    def _():
        side_body(wid, users, (t_eu, t_mu, t_fu), (o_eu, o_mu, o_fu))

    @pl.when(wid >= 16)
    def _():
        side_body(wid - 16, items, (t_ei, t_mi, t_fi), (o_ei, o_mi, o_fi))


@functools.cache
def _sc_gather():
    return pl.kernel(
        _sc_body,
        out_type=[jax.ShapeDtypeStruct((D, B), jnp.float32)] * 6,
        mesh=plsc.VectorSubcoreMesh(core_axis_name="c", subcore_axis_name="s"),
        scratch_types=[
            pltpu.VMEM((B,), jnp.int32),
            pltpu.VMEM((N_ROWS,), jnp.float32),
            pltpu.VMEM((OSUB, OC), jnp.float32),
            pltpu.SemaphoreType.DMA,
        ],
        compiler_params=pltpu.CompilerParams(use_tc_tiling_on_sc=True,
                                             needs_layout_passes=False),
    )


BT = 4096


def _tc_body(eu, mu, fu, ei, mi, fi, w1aT, w1bT, b1, w2a, w2b, b2, out):
    def imask(m):
        return lax.bitcast_convert_type(m[...], jnp.int32).astype(jnp.float32)

    xu = eu[...] * imask(mu)
    xi = ei[...] * imask(mi)
    mf = fu[...] * fi[...]
    h = jnp.dot(w1aT[...], xu, preferred_element_type=jnp.float32)
    h = h + jnp.dot(w1bT[...], xi, preferred_element_type=jnp.float32)
    h = jnp.maximum(h + b1[...], 0.0)
    o = jnp.dot(w2a[...], h, preferred_element_type=jnp.float32)
    o = o + jnp.dot(w2b[...], mf, preferred_element_type=jnp.float32)
    out[...] = o + b2[0, 0]


_tc_call = pl.pallas_call(
    _tc_body,
    grid=(B // BT,),
    in_specs=[pl.BlockSpec((D, BT), lambda n: (0, n))] * 6 + [
        pl.BlockSpec((D, D), lambda n: (0, 0)),
        pl.BlockSpec((D, D), lambda n: (0, 0)),
        pl.BlockSpec((D, 1), lambda n: (0, 0)),
        pl.BlockSpec((1, D), lambda n: (0, 0)),
        pl.BlockSpec((1, D), lambda n: (0, 0)),
        pl.BlockSpec((1, 1), lambda n: (0, 0)),
    ],
    out_specs=pl.BlockSpec((1, BT), lambda n: (0, n)),
    out_shape=jax.ShapeDtypeStruct((1, B), jnp.float32),
)


def kernel(users, items, emb_user_mlp, emb_item_mlp, emb_user_mf, emb_item_mf,
           user_mask, item_mask, W1, b1, W2, b2):
    fbits = lambda m: lax.bitcast_convert_type(m, jnp.float32)
    eu, mu, fu, ei, mi, fi = _sc_gather()(
        users, items,
        emb_user_mlp.T, fbits(user_mask).T, emb_user_mf.T,
        emb_item_mlp.T, fbits(item_mask).T, emb_item_mf.T)
    o = _tc_call(eu, mu, fu, ei, mi, fi,
                 W1[:D].T, W1[D:].T, b1.reshape(D, 1),
                 W2[:D].reshape(1, D), W2[D:].reshape(1, D),
                 b2.reshape(1, 1))
    return o.reshape(B, 1)
